# Initial kernel scaffold; baseline (speedup 1.0000x reference)
import jax
import jax.numpy as jnp
from jax.experimental import pallas as pl


def kernel(query_vector, corpus_vectors, bm25_scores, k):
    raise NotImplementedError("write your pallas kernel here")


# speedup vs baseline: 2.5015x; 2.5015x over previous
import functools

import jax
import jax.numpy as jnp
from jax import lax
from jax.experimental import pallas as pl
from jax.experimental.pallas import tpu as pltpu
from jax.experimental.pallas import tpu_sc as plsc

K_ROWS = 1_000_000
D = 64
TOP_K = 5
W0, W1 = 0.9, 1.1
EPS = 1e-8
NEG = -3.0e38

BLK = 8192
GRID = (K_ROWS + BLK - 1) // BLK


def _scores_body(lhs_ref, corpus_ref, bm25_ref, out_ref):
    i = pl.program_id(0)
    lhs = lhs_ref[...]
    c = corpus_ref[...]
    dn = (((1,), (1,)), ((), ()))
    dots = lax.dot_general(lhs, c, dn, preferred_element_type=jnp.float32)[0:1, :]
    ss = lax.dot_general(lhs, c * c, dn, preferred_element_type=jnp.float32)[1:2, :]
    q = lhs[0:1, :]
    qn2 = jnp.sum(q * q)
    r = lax.rsqrt(jnp.maximum(ss * qn2, EPS * EPS))
    cos = dots * r
    b = bm25_ref[...]
    score = (W0 * b + W1 * cos) * (1.0 / (W0 + W1))
    flat = i * BLK + lax.broadcasted_iota(jnp.int32, (1, BLK), 1)
    out_ref[...] = jnp.where(flat < K_ROWS, score, NEG)


def _compute_scores(lhs, corpus_vectors, bm25_2d):
    return pl.pallas_call(
        _scores_body,
        grid=(GRID,),
        in_specs=[
            pl.BlockSpec((8, D), lambda i: (0, 0)),
            pl.BlockSpec((BLK, D), lambda i: (i, 0)),
            pl.BlockSpec((1, BLK), lambda i: (0, i)),
        ],
        out_specs=pl.BlockSpec((1, BLK), lambda i: (0, i)),
        out_shape=jax.ShapeDtypeStruct((1, K_ROWS), jnp.float32),
        compiler_params=pltpu.CompilerParams(
            dimension_semantics=("arbitrary",),
        ),
    )(lhs, corpus_vectors, bm25_2d)


NSUB = 16
CH = 62528
NG = CH // 16
LAST_BASE = K_ROWS - CH
OVERLAP_G = (15 * CH - LAST_BASE) // 16
UNROLL = 8

_sc_mesh = plsc.VectorSubcoreMesh(
    core_axis_name="c", subcore_axis_name="s", num_cores=1
)


def _lane_of(eq):
    ffs = plsc.all_reduce_ffs(eq)
    if ffs.ndim == 0:
        return ffs
    return jnp.max(ffs)


def _extract_max(av, ai):
    mx = jnp.max(av)
    lane = _lane_of(av == mx)
    lanes = lax.iota(jnp.int32, 16)
    eqf = lanes == jnp.full((16,), lane, jnp.int32)
    grp = jnp.max(jnp.where(eqf, ai, 0))
    return mx, lane, grp, eqf


def _topk_body(scores_hbm, corpus_hbm, rel_out, ids_out, shv, shi,
               sbuf, candv, candi, mv, mi,
               ids16, rows16, sem):
    wid = lax.axis_index("s")
    base = jnp.where(wid == NSUB - 1, LAST_BASE, wid * CH)
    base = pl.multiple_of(base, 8)
    pltpu.sync_copy(scores_hbm.at[pl.ds(base, CH)], sbuf)

    @pl.when(wid == NSUB - 1)
    def _mask_overlap():
        neg = jnp.full((16,), NEG, jnp.float32)
        for g in range(OVERLAP_G):
            sbuf[pl.ds(g * 16, 16)] = neg

    lanes = lax.iota(jnp.int32, 16)
    cv = jnp.full((16,), NEG, jnp.float32)
    ci = jnp.zeros((16,), jnp.int32)

    for j in range(TOP_K):
        def scan_body(go, carry):
            av, ai = carry
            for u in range(UNROLL):
                g = go * UNROLL + u
                v = sbuf[pl.ds(pl.multiple_of(g * 16, 16), 16)]
                m = v > av
                av = jnp.where(m, v, av)
                ai = jnp.where(m, jnp.full((16,), 1, jnp.int32) * g, ai)
            return av, ai

        av, ai = lax.fori_loop(
            0, NG // UNROLL, scan_body,
            (jnp.full((16,), NEG, jnp.float32), jnp.zeros((16,), jnp.int32)),
        )
        mx, lane, grp, eqf = _extract_max(av, ai)
        lidx = grp * 16 + lane
        cv = jnp.where(lanes == j, jnp.full((16,), mx, jnp.float32), cv)
        ci = jnp.where(lanes == j, jnp.full((16,), base + lidx, jnp.int32), ci)
        plsc.store_scatter(sbuf, [jnp.full((16,), lidx, jnp.int32)],
                           jnp.full((16,), NEG, jnp.float32), mask=eqf)

    candv[...] = cv
    candi[...] = ci
    pltpu.sync_copy(candv, shv.at[wid])
    pltpu.sync_copy(candi, shi.at[wid])
    plsc.subcore_barrier()

    @pl.when(wid == 0)
    def _merge():
        pltpu.sync_copy(shv, mv)
        pltpu.sync_copy(shi, mi)
        idvec = jnp.zeros((16,), jnp.int32)
        for j in range(TOP_K):
            av = jnp.full((16,), NEG, jnp.float32)
            ai = jnp.zeros((16,), jnp.int32)
            for g in range(NSUB):
                v = mv[g, :]
                m = v > av
                av = jnp.where(m, v, av)
                ai = jnp.where(m, jnp.full((16,), g, jnp.int32), ai)
            mx, lane, grp, eqf = _extract_max(av, ai)
            gid_row = mi[grp]
            gid = jnp.max(jnp.where(eqf, gid_row, 0))
            idvec = jnp.where(lanes == j, jnp.full((16,), gid, jnp.int32), idvec)
            plsc.store_scatter(
                mv, [jnp.full((16,), grp, jnp.int32), jnp.full((16,), lane, jnp.int32)],
                jnp.full((16,), NEG, jnp.float32), mask=eqf)
            pltpu.sync_copy(corpus_hbm.at[pl.ds(gid, 1), :], rows16.at[pl.ds(j, 1), :])
        ids16[...] = idvec
        pltpu.sync_copy(rows16.at[pl.ds(0, TOP_K + 3), :],
                        rel_out.at[pl.ds(0, TOP_K + 3), :])
        pltpu.sync_copy(ids16, ids_out)


_topk_call = functools.partial(
    pl.kernel,
    out_type=(
        jax.ShapeDtypeStruct((16, D), jnp.float32),
        jax.ShapeDtypeStruct((16,), jnp.int32),
        jax.ShapeDtypeStruct((NSUB, 16), jnp.float32),
        jax.ShapeDtypeStruct((NSUB, 16), jnp.int32),
    ),
    mesh=_sc_mesh,
    compiler_params=pltpu.CompilerParams(needs_layout_passes=False),
    scratch_types=[
        pltpu.VMEM((CH,), jnp.float32),
        pltpu.VMEM((16,), jnp.float32),
        pltpu.VMEM((16,), jnp.int32),
        pltpu.VMEM((NSUB, 16), jnp.float32),
        pltpu.VMEM((NSUB, 16), jnp.int32),
        pltpu.VMEM((16,), jnp.int32),
        pltpu.VMEM((16, D), jnp.float32),
        pltpu.SemaphoreType.DMA,
    ],
)(_topk_body)


def kernel(query_vector, corpus_vectors, bm25_scores, k):
    del k
    lhs = jnp.zeros((8, D), jnp.float32).at[0].set(query_vector).at[1].set(1.0)
    bm25_2d = bm25_scores.reshape(1, K_ROWS)
    scores2d = _compute_scores(lhs, corpus_vectors, bm25_2d)
    scores = scores2d.reshape(K_ROWS)
    rel16, ids16, _, _ = _topk_call(scores, corpus_vectors)
    return rel16[:TOP_K], query_vector, scores, ids16[:TOP_K]


# speedup vs baseline: 6.8013x; 2.7189x over previous
import functools

import jax
import jax.numpy as jnp
from jax import lax
from jax.experimental import pallas as pl
from jax.experimental.pallas import tpu as pltpu
from jax.experimental.pallas import tpu_sc as plsc

K_ROWS = 1_000_000
D = 64
TOP_K = 5
W0, W1 = 0.9, 1.1
EPS = 1e-8
NEG = -3.0e38

BLK = 8192
GRID = (K_ROWS + BLK - 1) // BLK


def _scores_body(lhs_ref, corpus_ref, bm25_ref, out_ref):
    i = pl.program_id(0)
    lhs = lhs_ref[...]
    c = corpus_ref[...]
    dn = (((1,), (0,)), ((), ()))
    dots = lax.dot_general(lhs, c, dn, preferred_element_type=jnp.float32)[0:1, :]
    ss = lax.dot_general(lhs, c * c, dn, preferred_element_type=jnp.float32)[1:2, :]
    q = lhs[0:1, :]
    qn2 = jnp.sum(q * q)
    r = lax.rsqrt(jnp.maximum(ss * qn2, EPS * EPS))
    cos = dots * r
    b = bm25_ref[...]
    score = (W0 * b + W1 * cos) * (1.0 / (W0 + W1))
    flat = i * BLK + lax.broadcasted_iota(jnp.int32, (1, BLK), 1)
    out_ref[...] = jnp.where(flat < K_ROWS, score, NEG)


def _compute_scores(lhs, corpus_t, bm25_2d):
    return pl.pallas_call(
        _scores_body,
        grid=(GRID,),
        in_specs=[
            pl.BlockSpec((8, D), lambda i: (0, 0)),
            pl.BlockSpec((D, BLK), lambda i: (0, i)),
            pl.BlockSpec((1, BLK), lambda i: (0, i)),
        ],
        out_specs=pl.BlockSpec((1, BLK), lambda i: (0, i)),
        out_shape=jax.ShapeDtypeStruct((1, K_ROWS), jnp.float32),
        compiler_params=pltpu.CompilerParams(
            dimension_semantics=("arbitrary",),
        ),
    )(lhs, corpus_t, bm25_2d)


NSUB = 16
CH = 62528
NG = CH // 16
LAST_BASE = K_ROWS - CH
OVERLAP_G = (15 * CH - LAST_BASE) // 16
UNROLL = 8

_sc_mesh = plsc.VectorSubcoreMesh(
    core_axis_name="c", subcore_axis_name="s", num_cores=1
)


def _lane_of(eq):
    ffs = plsc.all_reduce_ffs(eq)
    if ffs.ndim == 0:
        return ffs
    return jnp.max(ffs)


def _extract_max(av, ai):
    mx = jnp.max(av)
    lane = _lane_of(av == mx)
    lanes = lax.iota(jnp.int32, 16)
    eqf = lanes == jnp.full((16,), lane, jnp.int32)
    grp = jnp.max(jnp.where(eqf, ai, 0))
    return mx, lane, grp, eqf


def _topk_body(scores_hbm, corpus_t_hbm, rel_flat_out, ids_out, shv, shi,
               sbuf, candv, candi, mv, mi,
               ids16, stripe, rowbuf, sem):
    wid = lax.axis_index("s")
    base = jnp.where(wid == NSUB - 1, LAST_BASE, wid * CH)
    base = pl.multiple_of(base, 8)
    pltpu.sync_copy(scores_hbm.at[pl.ds(base, CH)], sbuf)

    @pl.when(wid == NSUB - 1)
    def _mask_overlap():
        neg = jnp.full((16,), NEG, jnp.float32)
        for g in range(OVERLAP_G):
            sbuf[pl.ds(g * 16, 16)] = neg

    lanes = lax.iota(jnp.int32, 16)
    cv = jnp.full((16,), NEG, jnp.float32)
    ci = jnp.zeros((16,), jnp.int32)

    for j in range(TOP_K):
        def scan_body(go, carry):
            av, ai = carry
            for u in range(UNROLL):
                g = go * UNROLL + u
                v = sbuf[pl.ds(pl.multiple_of(g * 16, 16), 16)]
                m = v > av
                av = jnp.where(m, v, av)
                ai = jnp.where(m, jnp.full((16,), 1, jnp.int32) * g, ai)
            return av, ai

        av, ai = lax.fori_loop(
            0, NG // UNROLL, scan_body,
            (jnp.full((16,), NEG, jnp.float32), jnp.zeros((16,), jnp.int32)),
        )
        mx, lane, grp, eqf = _extract_max(av, ai)
        lidx = grp * 16 + lane
        cv = jnp.where(lanes == j, jnp.full((16,), mx, jnp.float32), cv)
        ci = jnp.where(lanes == j, jnp.full((16,), base + lidx, jnp.int32), ci)
        plsc.store_scatter(sbuf, [jnp.full((16,), lidx, jnp.int32)],
                           jnp.full((16,), NEG, jnp.float32), mask=eqf)

    candv[...] = cv
    candi[...] = ci
    pltpu.sync_copy(candv, shv.at[wid])
    pltpu.sync_copy(candi, shi.at[wid])
    plsc.subcore_barrier()

    @pl.when(wid == 0)
    def _merge():
        pltpu.sync_copy(shv, mv)
        pltpu.sync_copy(shi, mi)
        idvec = jnp.zeros((16,), jnp.int32)
        for j in range(TOP_K):
            av = jnp.full((16,), NEG, jnp.float32)
            ai = jnp.zeros((16,), jnp.int32)
            for g in range(NSUB):
                v = mv[g, :]
                m = v > av
                av = jnp.where(m, v, av)
                ai = jnp.where(m, jnp.full((16,), g, jnp.int32), ai)
            mx, lane, grp, eqf = _extract_max(av, ai)
            gid_row = mi[grp]
            gid = jnp.max(jnp.where(eqf, gid_row, 0))
            idvec = jnp.where(lanes == j, jnp.full((16,), gid, jnp.int32), idvec)
            plsc.store_scatter(
                mv, [jnp.full((16,), grp, jnp.int32), jnp.full((16,), lane, jnp.int32)],
                jnp.full((16,), NEG, jnp.float32), mask=eqf)
            col = jnp.remainder(gid, 128)
            base128 = pl.multiple_of(gid - col, 128)
            pltpu.sync_copy(corpus_t_hbm.at[:, pl.ds(base128, 128)], stripe)
            cols = jnp.full((16,), col, jnp.int32)
            for kk in range(D // 16):
                rows = lax.iota(jnp.int32, 16) + 16 * kk
                rowbuf[pl.ds(16 * kk, 16)] = plsc.load_gather(stripe, [rows, cols])
            pltpu.sync_copy(rowbuf, rel_flat_out.at[pl.ds(j * D, D)])
        ids16[...] = idvec
        pltpu.sync_copy(ids16, ids_out)


_topk_call = functools.partial(
    pl.kernel,
    out_type=(
        jax.ShapeDtypeStruct((16 * D,), jnp.float32),
        jax.ShapeDtypeStruct((16,), jnp.int32),
        jax.ShapeDtypeStruct((NSUB, 16), jnp.float32),
        jax.ShapeDtypeStruct((NSUB, 16), jnp.int32),
    ),
    mesh=_sc_mesh,
    compiler_params=pltpu.CompilerParams(needs_layout_passes=False),
    scratch_types=[
        pltpu.VMEM((CH,), jnp.float32),
        pltpu.VMEM((16,), jnp.float32),
        pltpu.VMEM((16,), jnp.int32),
        pltpu.VMEM((NSUB, 16), jnp.float32),
        pltpu.VMEM((NSUB, 16), jnp.int32),
        pltpu.VMEM((16,), jnp.int32),
        pltpu.VMEM((D, 128), jnp.float32),
        pltpu.VMEM((D,), jnp.float32),
        pltpu.SemaphoreType.DMA,
    ],
)(_topk_body)


def kernel(query_vector, corpus_vectors, bm25_scores, k):
    del k
    corpus_t = corpus_vectors.T
    lhs = jnp.zeros((8, D), jnp.float32).at[0].set(query_vector).at[1].set(1.0)
    bm25_2d = bm25_scores.reshape(1, K_ROWS)
    scores2d = _compute_scores(lhs, corpus_t, bm25_2d)
    scores = scores2d.reshape(K_ROWS)
    rel_flat, ids16, _, _ = _topk_call(scores, corpus_t)
    rel = rel_flat.reshape(16, D)[:TOP_K]
    return rel, query_vector, scores, ids16[:TOP_K]


# speedup vs baseline: 12.4332x; 1.8281x over previous
import functools

import jax
import jax.numpy as jnp
from jax import lax
from jax.experimental import pallas as pl
from jax.experimental.pallas import tpu as pltpu
from jax.experimental.pallas import tpu_sc as plsc

K_ROWS = 1_000_000
D = 64
TOP_K = 5
W0, W1 = 0.9, 1.1
EPS = 1e-8
NEG = -3.0e38

BLK = 32768
GRID = (K_ROWS + BLK - 1) // BLK
SCALE_Q = W1 / (W0 + W1)
SCALE_B = W0 / (W0 + W1)


def _scores_body(lhs_ref, corpus_ref, bm25_ref, out_ref):
    i = pl.program_id(0)
    lhs = lhs_ref[...]
    c = corpus_ref[...]
    dn = (((1,), (0,)), ((), ()))
    dots = lax.dot_general(lhs, c, dn, preferred_element_type=jnp.float32)[0, :]
    ss = lax.dot_general(lhs, c * c, dn, preferred_element_type=jnp.float32)[1, :]
    q = lhs[0, :]
    qn2 = jnp.sum(q * q) * (1.0 / (SCALE_Q * SCALE_Q))
    r = lax.rsqrt(jnp.maximum(ss * qn2, EPS * EPS))
    b = bm25_ref[...]
    score = SCALE_B * b + dots * r
    out_ref[...] = score

    @pl.when(i == GRID - 1)
    def _tail():
        flat = i * BLK + lax.iota(jnp.int32, BLK)
        out_ref[...] = jnp.where(flat < K_ROWS, score, NEG)


def _compute_scores(lhs, corpus_t, bm25):
    return pl.pallas_call(
        _scores_body,
        grid=(GRID,),
        in_specs=[
            pl.BlockSpec((8, D), lambda i: (0, 0)),
            pl.BlockSpec((D, BLK), lambda i: (0, i)),
            pl.BlockSpec((BLK,), lambda i: (i,)),
        ],
        out_specs=pl.BlockSpec((BLK,), lambda i: (i,)),
        out_shape=jax.ShapeDtypeStruct((K_ROWS,), jnp.float32),
        compiler_params=pltpu.CompilerParams(
            dimension_semantics=("arbitrary",),
        ),
    )(lhs, corpus_t, bm25)


NSUB = 16
CH = 62528
NG = CH // 16
LAST_BASE = K_ROWS - CH
OVERLAP_G = (15 * CH - LAST_BASE) // 16
UNROLL = 8

_sc_mesh = plsc.VectorSubcoreMesh(
    core_axis_name="c", subcore_axis_name="s", num_cores=1
)


def _lane_of(eq):
    ffs = plsc.all_reduce_ffs(eq)
    if ffs.ndim == 0:
        return ffs
    return jnp.max(ffs)


def _extract_max(av, ai):
    mx = jnp.max(av)
    lane = _lane_of(av == mx)
    lanes = lax.iota(jnp.int32, 16)
    eqf = lanes == jnp.full((16,), lane, jnp.int32)
    grp = jnp.max(jnp.where(eqf, ai, 0))
    return mx, lane, grp, eqf


def _topk_body(scores_hbm, corpus_t_hbm, rel_flat_out, ids_out, shv, shi,
               sbuf, candv, candi, mv, mi,
               ids16, stripe, rowbuf, sem):
    wid = lax.axis_index("s")
    base = jnp.where(wid == NSUB - 1, LAST_BASE, wid * CH)
    base = pl.multiple_of(base, 8)
    pltpu.sync_copy(scores_hbm.at[pl.ds(base, CH)], sbuf)

    @pl.when(wid == NSUB - 1)
    def _mask_overlap():
        neg = jnp.full((16,), NEG, jnp.float32)
        for g in range(OVERLAP_G):
            sbuf[pl.ds(g * 16, 16)] = neg

    lanes = lax.iota(jnp.int32, 16)
    cv = jnp.full((16,), NEG, jnp.float32)
    ci = jnp.zeros((16,), jnp.int32)

    for j in range(TOP_K):
        def scan_body(go, carry):
            av, ai = carry
            for u in range(UNROLL):
                g = go * UNROLL + u
                v = sbuf[pl.ds(pl.multiple_of(g * 16, 16), 16)]
                m = v > av
                av = jnp.where(m, v, av)
                ai = jnp.where(m, jnp.full((16,), 1, jnp.int32) * g, ai)
            return av, ai

        av, ai = lax.fori_loop(
            0, NG // UNROLL, scan_body,
            (jnp.full((16,), NEG, jnp.float32), jnp.zeros((16,), jnp.int32)),
        )
        mx, lane, grp, eqf = _extract_max(av, ai)
        lidx = grp * 16 + lane
        cv = jnp.where(lanes == j, jnp.full((16,), mx, jnp.float32), cv)
        ci = jnp.where(lanes == j, jnp.full((16,), base + lidx, jnp.int32), ci)
        plsc.store_scatter(sbuf, [jnp.full((16,), lidx, jnp.int32)],
                           jnp.full((16,), NEG, jnp.float32), mask=eqf)

    candv[...] = cv
    candi[...] = ci
    pltpu.sync_copy(candv, shv.at[wid])
    pltpu.sync_copy(candi, shi.at[wid])
    plsc.subcore_barrier()

    @pl.when(wid == 0)
    def _merge():
        pltpu.sync_copy(shv, mv)
        pltpu.sync_copy(shi, mi)
        idvec = jnp.zeros((16,), jnp.int32)
        for j in range(TOP_K):
            av = jnp.full((16,), NEG, jnp.float32)
            ai = jnp.zeros((16,), jnp.int32)
            for g in range(NSUB):
                v = mv[g, :]
                m = v > av
                av = jnp.where(m, v, av)
                ai = jnp.where(m, jnp.full((16,), g, jnp.int32), ai)
            mx, lane, grp, eqf = _extract_max(av, ai)
            gid_row = mi[grp]
            gid = jnp.max(jnp.where(eqf, gid_row, 0))
            idvec = jnp.where(lanes == j, jnp.full((16,), gid, jnp.int32), idvec)
            plsc.store_scatter(
                mv, [jnp.full((16,), grp, jnp.int32), jnp.full((16,), lane, jnp.int32)],
                jnp.full((16,), NEG, jnp.float32), mask=eqf)
            col = jnp.remainder(gid, 128)
            base128 = pl.multiple_of(gid - col, 128)
            pltpu.sync_copy(corpus_t_hbm.at[:, pl.ds(base128, 128)], stripe)
            cols = jnp.full((16,), col, jnp.int32)
            for kk in range(D // 16):
                rows = lax.iota(jnp.int32, 16) + 16 * kk
                rowbuf[pl.ds(16 * kk, 16)] = plsc.load_gather(stripe, [rows, cols])
            pltpu.sync_copy(rowbuf, rel_flat_out.at[pl.ds(j * D, D)])
        ids16[...] = idvec
        pltpu.sync_copy(ids16, ids_out)


_topk_call = functools.partial(
    pl.kernel,
    out_type=(
        jax.ShapeDtypeStruct((16 * D,), jnp.float32),
        jax.ShapeDtypeStruct((16,), jnp.int32),
        jax.ShapeDtypeStruct((NSUB, 16), jnp.float32),
        jax.ShapeDtypeStruct((NSUB, 16), jnp.int32),
    ),
    mesh=_sc_mesh,
    compiler_params=pltpu.CompilerParams(needs_layout_passes=False),
    scratch_types=[
        pltpu.VMEM((CH,), jnp.float32),
        pltpu.VMEM((16,), jnp.float32),
        pltpu.VMEM((16,), jnp.int32),
        pltpu.VMEM((NSUB, 16), jnp.float32),
        pltpu.VMEM((NSUB, 16), jnp.int32),
        pltpu.VMEM((16,), jnp.int32),
        pltpu.VMEM((D, 128), jnp.float32),
        pltpu.VMEM((D,), jnp.float32),
        pltpu.SemaphoreType.DMA,
    ],
)(_topk_body)


def kernel(query_vector, corpus_vectors, bm25_scores, k):
    del k
    corpus_t = corpus_vectors.T
    lhs = (jnp.zeros((8, D), jnp.float32)
           .at[0].set(query_vector * SCALE_Q).at[1].set(1.0))
    scores = _compute_scores(lhs, corpus_t, bm25_scores)
    rel_flat, ids16, _, _ = _topk_call(scores, corpus_t)
    rel = rel_flat.reshape(16, D)[:TOP_K]
    return rel, query_vector, scores, ids16[:TOP_K]


# speedup vs baseline: 13.5516x; 1.0900x over previous
import functools

import jax
import jax.numpy as jnp
from jax import lax
from jax.experimental import pallas as pl
from jax.experimental.pallas import tpu as pltpu
from jax.experimental.pallas import tpu_sc as plsc

K_ROWS = 1_000_000
D = 64
TOP_K = 5
W0, W1 = 0.9, 1.1
EPS = 1e-8
NEG = -3.0e38

BLK = 32768
GRID = (K_ROWS + BLK - 1) // BLK
SCALE_Q = W1 / (W0 + W1)
SCALE_B = W0 / (W0 + W1)


def _scores_body(lhs_ref, corpus_ref, bm25_ref, out_ref):
    i = pl.program_id(0)
    lhs = lhs_ref[...]
    c = corpus_ref[...]
    dn = (((1,), (0,)), ((), ()))
    dots = lax.dot_general(lhs, c, dn, preferred_element_type=jnp.float32)[0, :]
    ss = lax.dot_general(lhs, c * c, dn, preferred_element_type=jnp.float32)[1, :]
    q = lhs[0, :]
    qn2 = jnp.sum(q * q) * (1.0 / (SCALE_Q * SCALE_Q))
    r = lax.rsqrt(jnp.maximum(ss * qn2, EPS * EPS))
    b = bm25_ref[...]
    score = SCALE_B * b + dots * r
    out_ref[...] = score

    @pl.when(i == GRID - 1)
    def _tail():
        flat = i * BLK + lax.iota(jnp.int32, BLK)
        out_ref[...] = jnp.where(flat < K_ROWS, score, NEG)


def _compute_scores(lhs, corpus_t, bm25):
    return pl.pallas_call(
        _scores_body,
        grid=(GRID,),
        in_specs=[
            pl.BlockSpec((8, D), lambda i: (0, 0)),
            pl.BlockSpec((D, BLK), lambda i: (0, i)),
            pl.BlockSpec((BLK,), lambda i: (i,)),
        ],
        out_specs=pl.BlockSpec((BLK,), lambda i: (i,)),
        out_shape=jax.ShapeDtypeStruct((K_ROWS,), jnp.float32),
        compiler_params=pltpu.CompilerParams(
            dimension_semantics=("arbitrary",),
        ),
    )(lhs, corpus_t, bm25)


NSUB = 16
CH = 62720
NG = CH // 16
LAST_BASE = K_ROWS - CH
OVERLAP_G = (15 * CH - LAST_BASE) // 16
UNROLL = 8

_sc_mesh = plsc.VectorSubcoreMesh(
    core_axis_name="c", subcore_axis_name="s", num_cores=1
)


def _lane_of(eq):
    ffs = plsc.all_reduce_ffs(eq)
    if ffs.ndim == 0:
        return ffs
    return jnp.max(ffs)


def _extract_max(av, ai):
    mx = jnp.max(av)
    lane = _lane_of(av == mx)
    lanes = lax.iota(jnp.int32, 16)
    eqf = lanes == jnp.full((16,), lane, jnp.int32)
    grp = jnp.max(jnp.where(eqf, ai, 0))
    return mx, lane, grp, eqf


def _topk_body(scores_hbm, corpus_t_hbm, rel_flat_out, ids_out, shv, shi,
               sbuf, candv, candi, mv, mi,
               ids16, stripe, rowbuf, sem):
    wid = lax.axis_index("s")
    base = jnp.where(wid == NSUB - 1, LAST_BASE, wid * CH)
    base = pl.multiple_of(base, 8)
    pltpu.sync_copy(scores_hbm.at[pl.ds(base, CH)], sbuf)

    @pl.when(wid == NSUB - 1)
    def _mask_overlap():
        neg = jnp.full((16,), NEG, jnp.float32)
        for g in range(OVERLAP_G):
            sbuf[pl.ds(g * 16, 16)] = neg

    lanes = lax.iota(jnp.int32, 16)
    cv = jnp.full((16,), NEG, jnp.float32)
    ci = jnp.zeros((16,), jnp.int32)

    def scan_body(go, carry):
        av, ai = carry
        for u in range(UNROLL):
            g = go * UNROLL + u
            v = sbuf[pl.ds(pl.multiple_of(g * 16, 16), 16)]
            m = v > av
            av = jnp.where(m, v, av)
            ai = jnp.where(m, jnp.full((16,), 1, jnp.int32) * g, ai)
        return av, ai

    av, ai = lax.fori_loop(
        0, NG // UNROLL, scan_body,
        (jnp.full((16,), NEG, jnp.float32), jnp.zeros((16,), jnp.int32)),
    )

    for j in range(TOP_K):
        mx, lane, grp, eqf = _extract_max(av, ai)
        lidx = grp * 16 + lane
        cv = jnp.where(lanes == j, jnp.full((16,), mx, jnp.float32), cv)
        ci = jnp.where(lanes == j, jnp.full((16,), base + lidx, jnp.int32), ci)
        if j < TOP_K - 1:
            plsc.store_scatter(sbuf, [jnp.full((16,), lidx, jnp.int32)],
                               jnp.full((16,), NEG, jnp.float32), mask=eqf)
            lane_v = jnp.full((16,), lane, jnp.int32)

            def rescan_body(go, carry):
                rv, ri = carry
                gidx = go * 16 + lanes
                v = plsc.load_gather(sbuf, [gidx * 16 + lane_v])
                m = v > rv
                rv = jnp.where(m, v, rv)
                ri = jnp.where(m, gidx, ri)
                return rv, ri

            rv, ri = lax.fori_loop(
                0, NG // 16, rescan_body,
                (jnp.full((16,), NEG, jnp.float32), jnp.zeros((16,), jnp.int32)),
            )
            mx2 = jnp.max(rv)
            l2 = _lane_of(rv == mx2)
            eq2 = lanes == jnp.full((16,), l2, jnp.int32)
            g2 = jnp.max(jnp.where(eq2, ri, 0))
            av = jnp.where(eqf, jnp.full((16,), mx2, jnp.float32), av)
            ai = jnp.where(eqf, jnp.full((16,), g2, jnp.int32), ai)

    candv[...] = cv
    candi[...] = ci
    pltpu.sync_copy(candv, shv.at[wid])
    pltpu.sync_copy(candi, shi.at[wid])
    plsc.subcore_barrier()

    @pl.when(wid == 0)
    def _merge():
        pltpu.sync_copy(shv, mv)
        pltpu.sync_copy(shi, mi)
        idvec = jnp.zeros((16,), jnp.int32)
        for j in range(TOP_K):
            av = jnp.full((16,), NEG, jnp.float32)
            ai = jnp.zeros((16,), jnp.int32)
            for g in range(NSUB):
                v = mv[g, :]
                m = v > av
                av = jnp.where(m, v, av)
                ai = jnp.where(m, jnp.full((16,), g, jnp.int32), ai)
            mx, lane, grp, eqf = _extract_max(av, ai)
            gid_row = mi[grp]
            gid = jnp.max(jnp.where(eqf, gid_row, 0))
            idvec = jnp.where(lanes == j, jnp.full((16,), gid, jnp.int32), idvec)
            plsc.store_scatter(
                mv, [jnp.full((16,), grp, jnp.int32), jnp.full((16,), lane, jnp.int32)],
                jnp.full((16,), NEG, jnp.float32), mask=eqf)
            col = jnp.remainder(gid, 128)
            base128 = pl.multiple_of(gid - col, 128)
            pltpu.sync_copy(corpus_t_hbm.at[:, pl.ds(base128, 128)], stripe)
            cols = jnp.full((16,), col, jnp.int32)
            for kk in range(D // 16):
                rows = lax.iota(jnp.int32, 16) + 16 * kk
                rowbuf[pl.ds(16 * kk, 16)] = plsc.load_gather(stripe, [rows, cols])
            pltpu.sync_copy(rowbuf, rel_flat_out.at[pl.ds(j * D, D)])
        ids16[...] = idvec
        pltpu.sync_copy(ids16, ids_out)


_topk_call = functools.partial(
    pl.kernel,
    out_type=(
        jax.ShapeDtypeStruct((16 * D,), jnp.float32),
        jax.ShapeDtypeStruct((16,), jnp.int32),
        jax.ShapeDtypeStruct((NSUB, 16), jnp.float32),
        jax.ShapeDtypeStruct((NSUB, 16), jnp.int32),
    ),
    mesh=_sc_mesh,
    compiler_params=pltpu.CompilerParams(needs_layout_passes=False),
    scratch_types=[
        pltpu.VMEM((CH,), jnp.float32),
        pltpu.VMEM((16,), jnp.float32),
        pltpu.VMEM((16,), jnp.int32),
        pltpu.VMEM((NSUB, 16), jnp.float32),
        pltpu.VMEM((NSUB, 16), jnp.int32),
        pltpu.VMEM((16,), jnp.int32),
        pltpu.VMEM((D, 128), jnp.float32),
        pltpu.VMEM((D,), jnp.float32),
        pltpu.SemaphoreType.DMA,
    ],
)(_topk_body)


def kernel(query_vector, corpus_vectors, bm25_scores, k):
    del k
    corpus_t = corpus_vectors.T
    lhs = (jnp.zeros((8, D), jnp.float32)
           .at[0].set(query_vector * SCALE_Q).at[1].set(1.0))
    scores = _compute_scores(lhs, corpus_t, bm25_scores)
    rel_flat, ids16, _, _ = _topk_call(scores, corpus_t)
    rel = rel_flat.reshape(16, D)[:TOP_K]
    return rel, query_vector, scores, ids16[:TOP_K]
